# baseline (speedup 1.0000x reference)
import jax
import jax.numpy as jnp
from jax.experimental import pallas as pl
from jax.experimental.pallas import tpu as pltpu

NUM_HOPS = 2


def _mlp_body(pooled_ref, wc1_ref, bc1_ref, wc2_ref, bc2_ref, out_ref):
    h = jnp.maximum(pooled_ref[...] @ wc1_ref[...] + bc1_ref[...], 0.0)
    out_ref[...] = h @ wc2_ref[...] + bc2_ref[...]


def kernel(edge_index, node_features, pred_edges, emb_table, W1, b1, W2, b2, W3, b3,
           g1, be1, g2, be2, g3, be3, Wc1, bc1, Wc2, bc2):
    num_nodes = node_features.shape[0]
    num_graphs = pred_edges.shape[1]
    row, col = edge_index[0], edge_index[1]

    def khop_mask(s, t):
        m = jnp.zeros((num_nodes,), dtype=bool).at[s].set(True).at[t].set(True)
        for _ in range(NUM_HOPS):
            hit = jnp.zeros((num_nodes,), dtype=jnp.int32).at[row].add(
                m[col].astype(jnp.int32))
            m = m | (hit > 0)
        return m

    masks = jax.vmap(khop_mask)(pred_edges[0], pred_edges[1])
    evalid = masks[:, row] & masks[:, col]

    def bfs(start, ev):
        dist = jnp.full((num_nodes,), -1, dtype=jnp.int32).at[start].set(0)
        frontier = jnp.zeros((num_nodes,), dtype=bool).at[start].set(True)
        visited = frontier
        for d in range(1, NUM_HOPS + 1):
            fwd = jnp.zeros((num_nodes,), dtype=jnp.int32).at[col].add(
                (frontier[row] & ev).astype(jnp.int32))
            bwd = jnp.zeros((num_nodes,), dtype=jnp.int32).at[row].add(
                (frontier[col] & ev).astype(jnp.int32))
            nxt = ((fwd + bwd) > 0) & (~visited)
            dist = jnp.where(nxt, d, dist)
            visited = visited | nxt
            frontier = nxt
        return dist

    d1 = jax.vmap(bfs)(pred_edges[0], evalid)
    d2 = jax.vmap(bfs)(pred_edges[1], evalid)
    hp1 = NUM_HOPS + 1
    d1 = jnp.where(d1 == -1, hp1, d1)
    d2 = jnp.where(d2 == -1, hp1, d2)
    labels = 1 + jnp.minimum(d1, d2) + hp1 * jnp.minimum(jnp.maximum(d1, d2), hp1)

    mf = masks.astype(node_features.dtype)
    x = jnp.concatenate([
        jnp.broadcast_to(node_features, (num_graphs,) + node_features.shape),
        emb_table[labels]], axis=-1) * mf[..., None]

    n_tot = jnp.sum(mf)

    def gcn(x, W, b):
        def per_graph(args):
            xj, evj, mj = args
            xw = xj @ W
            ef = evj.astype(xj.dtype)
            deg = jnp.zeros((num_nodes,), dtype=xj.dtype).at[col].add(ef) + 1.0
            dinv = 1.0 / jnp.sqrt(jnp.clip(deg, 1.0))
            norm = dinv[row] * dinv[col] * ef
            out = jnp.zeros((num_nodes, W.shape[1]), dtype=xj.dtype).at[col].add(
                xw[row] * norm[:, None])
            out = out + xw * (dinv * dinv)[:, None]
            return (out + b) * mj[:, None]
        return jax.lax.map(per_graph, (x, evalid, mf))

    def bn(x, g, be):
        m = jnp.sum(x, axis=(0, 1)) / n_tot
        v = jnp.sum(jnp.square(x - m) * mf[..., None], axis=(0, 1)) / n_tot
        return ((x - m) / jnp.sqrt(v + 1e-5) * g + be) * mf[..., None]

    convs = [(W1, b1, g1, be1), (W2, b2, g2, be2), (W3, b3, g3, be3)]
    for i, (W, b, g, be) in enumerate(convs):
        x = gcn(x, W, b)
        x = bn(x, g, be)
        if i < len(convs) - 1:
            x = jax.nn.relu(x)
    counts = jnp.sum(mf, axis=1)
    pooled = jnp.sum(x, axis=1) / counts[:, None]

    scores2d = pl.pallas_call(
        _mlp_body,
        out_shape=jax.ShapeDtypeStruct((num_graphs, 1), jnp.float32),
    )(pooled, Wc1, bc1[None, :], Wc2, bc2[None, :])
    return scores2d[:, 0]


# speedup vs baseline: 52.7634x; 52.7634x over previous
import functools

import jax
import jax.numpy as jnp
from jax import lax
from jax.experimental import pallas as pl
from jax.experimental.pallas import tpu as pltpu
from jax.experimental.pallas import tpu_sc as plsc

_NUM_HOPS = 2
_CAP = 1024
_ECAP = 4096

_pallas_call = pl.pallas_call
_PREC = jax.lax.Precision.HIGHEST


def _phase1_sc(row, col, pred, num_nodes, num_edges, num_graphs):
    n16 = num_nodes // 16
    ch = 16000
    nchunk = num_edges // ch
    ch16 = ch // 16
    mesh = plsc.VectorSubcoreMesh(core_axis_name="c", subcore_axis_name="s")

    @functools.partial(
        pl.kernel,
        out_type=[jax.ShapeDtypeStruct((num_graphs, num_nodes), jnp.int32),
                  jax.ShapeDtypeStruct((num_graphs, num_nodes), jnp.int32),
                  jax.ShapeDtypeStruct((num_graphs, num_nodes), jnp.float32)],
        mesh=mesh,
        scratch_types=[pltpu.VMEM((num_nodes,), jnp.int32),
                       pltpu.VMEM((num_nodes,), jnp.int32),
                       pltpu.VMEM((num_nodes,), jnp.int32),
                       pltpu.VMEM((num_nodes,), jnp.int32),
                       pltpu.VMEM((num_nodes,), jnp.int32),
                       pltpu.VMEM((num_nodes,), jnp.float32),
                       pltpu.VMEM((ch,), jnp.int32),
                       pltpu.VMEM((ch,), jnp.int32),
                       pltpu.VMEM((2 * num_graphs,), jnp.int32)],
        compiler_params=pltpu.CompilerParams(needs_layout_passes=False),
    )
    def k(row_hbm, col_hbm, pred_hbm, masks_hbm, labels_hbm, deg_hbm,
          m, dist1, dist2, acc1, acc2, degf, er, ec, predv):
        wid = lax.axis_index("s") * 2 + lax.axis_index("c")
        widv = jnp.full((16,), wid, jnp.int32)
        lane = lax.iota(jnp.int32, 16)
        lane0 = lane == 0
        zeros16 = jnp.zeros((16,), jnp.int32)
        ones16 = jnp.ones((16,), jnp.int32)
        neg16 = jnp.full((16,), -1, jnp.int32)
        fz16 = jnp.zeros((16,), jnp.float32)

        pltpu.sync_copy(pred_hbm, predv)
        sv = plsc.load_gather(predv, [widv])
        tv = plsc.load_gather(predv, [widv + num_graphs])

        def fill(ref, val):
            def b(i, _):
                ref[pl.ds(i * 16, 16)] = val
                return 0
            lax.fori_loop(0, n16, b, 0)

        fill(m, zeros16)
        fill(acc1, zeros16)
        fill(acc2, zeros16)
        fill(degf, fz16)
        plsc.store_scatter(m, [sv], ones16, mask=lane0)
        plsc.store_scatter(m, [tv], ones16, mask=lane0)

        def edge_sweep(body_fn):
            for cidx in range(nchunk):
                pltpu.sync_copy(row_hbm.at[pl.ds(cidx * ch, ch)], er)
                pltpu.sync_copy(col_hbm.at[pl.ds(cidx * ch, ch)], ec)

                def eb(g, _):
                    r = er[pl.ds(g * 16, 16)]
                    c = ec[pl.ds(g * 16, 16)]
                    body_fn(r, c)
                    return 0
                lax.fori_loop(0, ch16, eb, 0)

        def khop_body(r, c):
            mc = plsc.load_gather(m, [c])
            plsc.addupdate_scatter(acc1, [r], mc)

        for _ in range(_NUM_HOPS):
            edge_sweep(khop_body)

            def upd(i, _):
                sl = pl.ds(i * 16, 16)
                m[sl] = jnp.where(acc1[sl] > 0, ones16, m[sl])
                acc1[sl] = zeros16
                return 0
            lax.fori_loop(0, n16, upd, 0)

        fill(dist1, neg16)
        fill(dist2, neg16)
        plsc.store_scatter(dist1, [sv], zeros16, mask=lane0)
        plsc.store_scatter(dist2, [tv], zeros16, mask=lane0)

        for d in range(1, _NUM_HOPS + 1):
            dm1 = jnp.full((16,), d - 1, jnp.int32)
            first_hop = d == 1

            def bfs_body(r, c, dm1=dm1, first_hop=first_hop):
                mr = plsc.load_gather(m, [r])
                mc = plsc.load_gather(m, [c])
                ev = mr * mc
                dr1 = plsc.load_gather(dist1, [r])
                dc1 = plsc.load_gather(dist1, [c])
                dr2 = plsc.load_gather(dist2, [r])
                dc2 = plsc.load_gather(dist2, [c])
                plsc.addupdate_scatter(acc1, [c], jnp.where(dr1 == dm1, ev, zeros16))
                plsc.addupdate_scatter(acc1, [r], jnp.where(dc1 == dm1, ev, zeros16))
                plsc.addupdate_scatter(acc2, [c], jnp.where(dr2 == dm1, ev, zeros16))
                plsc.addupdate_scatter(acc2, [r], jnp.where(dc2 == dm1, ev, zeros16))
                if first_hop:
                    plsc.addupdate_scatter(degf, [c], ev.astype(jnp.float32))

            edge_sweep(bfs_body)
            dv = jnp.full((16,), d, jnp.int32)

            def updd(i, _, dv=dv):
                sl = pl.ds(i * 16, 16)
                d1v = dist1[sl]
                dist1[sl] = jnp.where((acc1[sl] > 0) & (d1v < 0), dv, d1v)
                acc1[sl] = zeros16
                d2v = dist2[sl]
                dist2[sl] = jnp.where((acc2[sl] > 0) & (d2v < 0), dv, d2v)
                acc2[sl] = zeros16
                return 0
            lax.fori_loop(0, n16, updd, 0)

        hp = jnp.full((16,), _NUM_HOPS + 1, jnp.int32)

        def labs(i, _):
            sl = pl.ds(i * 16, 16)
            a = dist1[sl]
            b = dist2[sl]
            a = jnp.where(a < 0, hp, a)
            b = jnp.where(b < 0, hp, b)
            mn = jnp.minimum(a, b)
            mx = jnp.minimum(jnp.maximum(a, b), hp)
            acc1[sl] = ones16 + mn + (_NUM_HOPS + 1) * mx
            return 0
        lax.fori_loop(0, n16, labs, 0)

        pltpu.sync_copy(m, masks_hbm.at[wid])
        pltpu.sync_copy(acc1, labels_hbm.at[wid])
        pltpu.sync_copy(degf, deg_hbm.at[wid])

    return k(row, col, pred.reshape(-1))


def _l1_body(x_ref, at_ref, w_ref, b_ref, vrow_ref, y_ref, s_ref, ss_ref):
    xw = jnp.dot(x_ref[0], w_ref[...], precision=_PREC,
                 preferred_element_type=jnp.float32)
    y = jnp.dot(at_ref[0], xw, precision=_PREC,
                preferred_element_type=jnp.float32)
    y = (y + b_ref[...]) * vrow_ref[0]
    y_ref[0] = y
    s_ref[0, 0] = jnp.sum(y, axis=0)
    ss_ref[0, 0] = jnp.sum(y * y, axis=0)


def _mid_body(y_ref, s_in_ref, ss_in_ref, g_ref, be_ref, ntot_ref,
              at_ref, w_ref, b_ref, vrow_ref, y_out_ref, s_ref, ss_ref):
    ntot = ntot_ref[0, 0]
    S = jnp.sum(s_in_ref[...], axis=0)
    SS = jnp.sum(ss_in_ref[...], axis=0)
    m = S / ntot
    v = (SS - S * S / ntot) / ntot
    scale = g_ref[0] / jnp.sqrt(v + 1e-5)
    shift = be_ref[0] - m * scale
    vr = vrow_ref[0]
    z = jnp.maximum((y_ref[0] * scale + shift) * vr, 0.0)
    xw = jnp.dot(z, w_ref[...], precision=_PREC,
                 preferred_element_type=jnp.float32)
    y = (jnp.dot(at_ref[0], xw, precision=_PREC,
                 preferred_element_type=jnp.float32) + b_ref[...]) * vr
    y_out_ref[0] = y
    s_ref[0, 0] = jnp.sum(y, axis=0)
    ss_ref[0, 0] = jnp.sum(y * y, axis=0)


def _readout_body(s3_ref, ss3_ref, cnt_ref, ntot_ref, g_ref, be_ref,
                  wc1_ref, bc1_ref, wc2_ref, bc2_ref, out_ref):
    ntot = ntot_ref[0, 0]
    S = jnp.sum(s3_ref[...], axis=0)
    SS = jnp.sum(ss3_ref[...], axis=0)
    m = S / ntot
    v = (SS - S * S / ntot) / ntot
    scale = g_ref[0] / jnp.sqrt(v + 1e-5)
    shift = be_ref[0] - m * scale
    cnt = cnt_ref[...]
    pooled = (s3_ref[...] * scale + shift * cnt) / cnt
    h = jnp.maximum(jnp.dot(pooled, wc1_ref[...], precision=_PREC,
                            preferred_element_type=jnp.float32) + bc1_ref[...],
                    0.0)
    out_ref[...] = jnp.dot(h, wc2_ref[...], precision=_PREC,
                           preferred_element_type=jnp.float32) + bc2_ref[...]


def _fast_path(x_cp, AT, vrow, cnt, n_tot, W1p, b1, W2, b2, W3, b3,
               g1, be1, g2, be2, g3, be3, Wc1, bc1, Wc2p, bc2p):
    G = x_cp.shape[0]
    H = W1p.shape[1]
    full = lambda shape: pl.BlockSpec(shape, lambda j: (0,) * len(shape))
    per_g3 = lambda shape: pl.BlockSpec(shape, lambda j: (j, 0, 0))
    vrow3 = vrow[:, :, None]

    y1, s1, ss1 = _pallas_call(
        _l1_body,
        grid=(G,),
        in_specs=[per_g3((1, _CAP, 256)), per_g3((1, _CAP, _CAP)),
                  full((256, H)), full((1, H)), per_g3((1, _CAP, 1))],
        out_specs=[per_g3((1, _CAP, H)), per_g3((1, 1, H)), per_g3((1, 1, H))],
        out_shape=[jax.ShapeDtypeStruct((G, _CAP, H), jnp.float32),
                   jax.ShapeDtypeStruct((G, 1, H), jnp.float32),
                   jax.ShapeDtypeStruct((G, 1, H), jnp.float32)],
    )(x_cp, AT, W1p, b1, vrow3)

    def mid(y, s_in, ss_in, g, be, W, b):
        return _pallas_call(
            _mid_body,
            grid=(G,),
            in_specs=[per_g3((1, _CAP, H)), full((G, H)), full((G, H)),
                      full((1, H)), full((1, H)), full((1, 1)),
                      per_g3((1, _CAP, _CAP)), full((H, H)), full((1, H)),
                      per_g3((1, _CAP, 1))],
            out_specs=[per_g3((1, _CAP, H)), per_g3((1, 1, H)), per_g3((1, 1, H))],
            out_shape=[jax.ShapeDtypeStruct((G, _CAP, H), jnp.float32),
                       jax.ShapeDtypeStruct((G, 1, H), jnp.float32),
                       jax.ShapeDtypeStruct((G, 1, H), jnp.float32)],
        )(y, s_in.reshape(G, H), ss_in.reshape(G, H), g, be, n_tot, AT, W, b,
          vrow3)

    y2, s2, ss2 = mid(y1, s1, ss1, g1, be1, W2, b2)
    y3, s3, ss3 = mid(y2, s2, ss2, g2, be2, W3, b3)
    s3 = s3.reshape(G, H)
    ss3 = ss3.reshape(G, H)

    full0 = lambda shape: pl.BlockSpec(shape, lambda: (0,) * len(shape))
    out = _pallas_call(
        _readout_body,
        in_specs=[full0((G, H)), full0((G, H)), full0((G, 1)), full0((1, 1)),
                  full0((1, H)), full0((1, H)), full0((H, 128)), full0((1, 128)),
                  full0((128, 128)), full0((1, 128))],
        out_specs=full0((G, 128)),
        out_shape=jax.ShapeDtypeStruct((G, 128), jnp.float32),
    )(s3, ss3, cnt, n_tot, g3, be3, Wc1, bc1, Wc2p, bc2p)
    return out[:, 0]


def kernel(edge_index, node_features, pred_edges, emb_table, W1, b1, W2, b2,
           W3, b3, g1, be1, g2, be2, g3, be3, Wc1, bc1, Wc2, bc2):
    num_nodes = node_features.shape[0]
    num_graphs = pred_edges.shape[1]
    feat = node_features.shape[1]
    row, col = edge_index[0], edge_index[1]

    num_edges = row.shape[0]
    if num_graphs == 32 and num_nodes % 16 == 0 and num_edges % 16000 == 0:
        masks_i, labels, degs = _phase1_sc(
            row, col, pred_edges, num_nodes, num_edges, num_graphs)
        masks = masks_i.astype(bool)
        evalid = masks[:, row] & masks[:, col]
        deg = degs + 1.0
    else:
        def khop_mask(s, t):
            m = jnp.zeros((num_nodes,), dtype=bool).at[s].set(True).at[t].set(True)
            for _ in range(_NUM_HOPS):
                hit = jnp.zeros((num_nodes,), dtype=jnp.int32).at[row].add(
                    m[col].astype(jnp.int32))
                m = m | (hit > 0)
            return m

        masks = jax.vmap(khop_mask)(pred_edges[0], pred_edges[1])
        evalid = masks[:, row] & masks[:, col]

        def bfs(start, ev):
            dist = jnp.full((num_nodes,), -1, dtype=jnp.int32).at[start].set(0)
            frontier = jnp.zeros((num_nodes,), dtype=bool).at[start].set(True)
            visited = frontier
            for d in range(1, _NUM_HOPS + 1):
                fwd = jnp.zeros((num_nodes,), dtype=jnp.int32).at[col].add(
                    (frontier[row] & ev).astype(jnp.int32))
                bwd = jnp.zeros((num_nodes,), dtype=jnp.int32).at[row].add(
                    (frontier[col] & ev).astype(jnp.int32))
                nxt = ((fwd + bwd) > 0) & (~visited)
                dist = jnp.where(nxt, d, dist)
                visited = visited | nxt
                frontier = nxt
            return dist

        d1 = jax.vmap(bfs)(pred_edges[0], evalid)
        d2 = jax.vmap(bfs)(pred_edges[1], evalid)
        hp1 = _NUM_HOPS + 1
        d1 = jnp.where(d1 == -1, hp1, d1)
        d2 = jnp.where(d2 == -1, hp1, d2)
        labels = 1 + jnp.minimum(d1, d2) + hp1 * jnp.minimum(
            jnp.maximum(d1, d2), hp1)

        ef = evalid.astype(jnp.float32)
        deg = jax.vmap(
            lambda e: jnp.zeros((num_nodes,), jnp.float32).at[col].add(e))(ef) + 1.0
    dinv = jax.lax.rsqrt(deg)

    nvalid = jnp.sum(masks.astype(jnp.int32), axis=1)
    nodes = jax.vmap(
        lambda mm: jnp.nonzero(mm, size=_CAP, fill_value=num_nodes)[0])(masks)
    nodes = nodes.astype(jnp.int32)
    validrow = (jnp.arange(_CAP)[None, :] < nvalid[:, None]).astype(jnp.float32)

    inv = jax.vmap(lambda nd: jnp.full((num_nodes + 1,), _CAP, jnp.int32)
                   .at[nd].set(jnp.arange(_CAP, dtype=jnp.int32)))(nodes)

    ecount = jnp.sum(evalid.astype(jnp.int32), axis=1)
    eidx = jax.vmap(
        lambda e: jnp.nonzero(e, size=_ECAP, fill_value=0)[0])(evalid)
    eidx = eidx.astype(jnp.int32)
    eflag = (jnp.arange(_ECAP)[None, :] < ecount[:, None]).astype(jnp.float32)

    row_e = row[eidx]
    col_e = col[eidx]
    norm = (jnp.take_along_axis(dinv, row_e, axis=1)
            * jnp.take_along_axis(dinv, col_e, axis=1) * eflag)
    r_c = jnp.take_along_axis(inv, row_e, axis=1)
    c_c = jnp.take_along_axis(inv, col_e, axis=1)

    dinv_pad = jnp.concatenate([dinv, jnp.ones((num_graphs, 1), jnp.float32)], 1)
    dvn = jnp.take_along_axis(dinv_pad, nodes, axis=1)
    dsq = dvn * dvn * validrow

    ar = jnp.arange(_CAP)

    def build_at(cc, rc, nm, ds):
        A = jnp.zeros((_CAP, _CAP), jnp.float32).at[cc, rc].add(
            nm, mode='drop')
        return A.at[ar, ar].add(ds)

    AT = jax.vmap(build_at)(c_c, r_c, norm, dsq)

    nf_pad = jnp.concatenate(
        [node_features, jnp.zeros((1, feat), jnp.float32)], 0)
    nf_c = nf_pad[nodes]
    lab_pad = jnp.concatenate(
        [labels, jnp.zeros((num_graphs, 1), labels.dtype)], 1)
    lab_c = jnp.take_along_axis(lab_pad, nodes, axis=1)
    emb_c = emb_table[lab_c]
    x_c = jnp.concatenate([nf_c, emb_c], -1) * validrow[..., None]
    in_dim = x_c.shape[-1]
    x_cp = jnp.pad(x_c, ((0, 0), (0, 0), (0, 256 - in_dim)))
    W1p = jnp.pad(W1, ((0, 256 - in_dim), (0, 0)))

    cnt = nvalid.astype(jnp.float32)[:, None]
    n_tot = jnp.sum(cnt).reshape(1, 1)
    Wc2p = jnp.pad(Wc2, ((0, 0), (0, 127)))
    bc2p = jnp.pad(bc2, (0, 127))[None, :]

    ok = (jnp.max(nvalid) <= _CAP) & (jnp.max(ecount) <= _ECAP)

    def fast():
        return _fast_path(x_cp, AT, validrow, cnt, n_tot, W1p, b1[None, :],
                          W2, b2[None, :], W3, b3[None, :], g1[None, :],
                          be1[None, :], g2[None, :], be2[None, :], g3[None, :],
                          be3[None, :], Wc1, bc1[None, :], Wc2p, bc2p)

    def slow():
        mf = masks.astype(node_features.dtype)
        x = jnp.concatenate([
            jnp.broadcast_to(node_features,
                             (num_graphs,) + node_features.shape),
            emb_table[labels]], axis=-1) * mf[..., None]
        ntot_s = jnp.sum(mf)

        def gcn(x, W, b):
            def per_graph(args):
                xj, evj, mj = args
                xw = xj @ W
                efj = evj.astype(xj.dtype)
                degj = jnp.zeros((num_nodes,), dtype=xj.dtype).at[col].add(efj) + 1.0
                dinvj = 1.0 / jnp.sqrt(jnp.clip(degj, 1.0))
                normj = dinvj[row] * dinvj[col] * efj
                outj = jnp.zeros((num_nodes, W.shape[1]), dtype=xj.dtype).at[col].add(
                    xw[row] * normj[:, None])
                outj = outj + xw * (dinvj * dinvj)[:, None]
                return (outj + b) * mj[:, None]
            return jax.lax.map(per_graph, (x, evalid, mf))

        def bn(x, g, be):
            m = jnp.sum(x, axis=(0, 1)) / ntot_s
            v = jnp.sum(jnp.square(x - m) * mf[..., None], axis=(0, 1)) / ntot_s
            return ((x - m) / jnp.sqrt(v + 1e-5) * g + be) * mf[..., None]

        xs = x
        for i, (W, b, g, be) in enumerate(
                [(W1, b1, g1, be1), (W2, b2, g2, be2), (W3, b3, g3, be3)]):
            xs = gcn(xs, W, b)
            xs = bn(xs, g, be)
            if i < 2:
                xs = jax.nn.relu(xs)
        counts = jnp.sum(mf, axis=1)
        pooled = jnp.sum(xs, axis=1) / counts[:, None]
        h = jax.nn.relu(pooled @ Wc1 + bc1)
        return (h @ Wc2 + bc2).squeeze(-1)

    return jax.lax.cond(ok, fast, slow)


# speedup vs baseline: 74.8507x; 1.4186x over previous
import functools

import jax
import jax.numpy as jnp
from jax import lax
from jax.experimental import pallas as pl
from jax.experimental.pallas import tpu as pltpu
from jax.experimental.pallas import tpu_sc as plsc

_NUM_HOPS = 2
_CAP = 1024
_ECAP = 4096

_pallas_call = pl.pallas_call
_PREC = jax.lax.Precision.HIGHEST


def _phase1_sc(row, col, pred, num_nodes, num_edges, num_graphs):
    n16 = num_nodes // 16
    ch = 16000
    nchunk = num_edges // ch
    ch16 = ch // 16
    mesh = plsc.VectorSubcoreMesh(core_axis_name="c", subcore_axis_name="s")

    @functools.partial(
        pl.kernel,
        out_type=[jax.ShapeDtypeStruct((num_graphs, num_nodes), jnp.int32),
                  jax.ShapeDtypeStruct((num_graphs, num_nodes), jnp.int32),
                  jax.ShapeDtypeStruct((num_graphs, num_nodes), jnp.float32),
                  jax.ShapeDtypeStruct((num_graphs, _ECAP), jnp.int32),
                  jax.ShapeDtypeStruct((num_graphs, _ECAP), jnp.int32),
                  jax.ShapeDtypeStruct((num_graphs, _CAP), jnp.int32),
                  jax.ShapeDtypeStruct((num_graphs, 16), jnp.int32)],
        mesh=mesh,
        scratch_types=[pltpu.VMEM((num_nodes,), jnp.int32),
                       pltpu.VMEM((num_nodes,), jnp.int32),
                       pltpu.VMEM((num_nodes,), jnp.int32),
                       pltpu.VMEM((num_nodes,), jnp.int32),
                       pltpu.VMEM((num_nodes,), jnp.int32),
                       pltpu.VMEM((num_nodes,), jnp.float32),
                       pltpu.VMEM((ch,), jnp.int32),
                       pltpu.VMEM((ch,), jnp.int32),
                       pltpu.VMEM((2 * num_graphs,), jnp.int32),
                       pltpu.VMEM((_ECAP,), jnp.int32),
                       pltpu.VMEM((_ECAP,), jnp.int32),
                       pltpu.VMEM((_CAP,), jnp.int32),
                       pltpu.VMEM((16,), jnp.int32)],
        compiler_params=pltpu.CompilerParams(needs_layout_passes=False),
    )
    def k(row_hbm, col_hbm, pred_hbm, masks_hbm, labels_hbm, deg_hbm,
          ero_hbm, eco_hbm, nodes_hbm, cnts_hbm,
          m, dist1, dist2, acc1, acc2, degf, er, ec, predv,
          ero, eco, nodeso, cntv):
        wid = lax.axis_index("s") * 2 + lax.axis_index("c")
        widv = jnp.full((16,), wid, jnp.int32)
        lane = lax.iota(jnp.int32, 16)
        lane0 = lane == 0
        zeros16 = jnp.zeros((16,), jnp.int32)
        ones16 = jnp.ones((16,), jnp.int32)
        neg16 = jnp.full((16,), -1, jnp.int32)
        fz16 = jnp.zeros((16,), jnp.float32)

        pltpu.sync_copy(pred_hbm, predv)
        sv = plsc.load_gather(predv, [widv])
        tv = plsc.load_gather(predv, [widv + num_graphs])

        def fill(ref, val, cnt):
            def b(i, _):
                ref[pl.ds(i * 16, 16)] = val
                return 0
            lax.fori_loop(0, cnt, b, 0)

        fill(m, zeros16, n16)
        fill(acc1, zeros16, n16)
        fill(acc2, zeros16, n16)
        fill(degf, fz16, n16)
        fill(ero, zeros16, _ECAP // 16)
        fill(eco, zeros16, _ECAP // 16)
        fill(nodeso, jnp.full((16,), num_nodes, jnp.int32), _CAP // 16)
        plsc.store_scatter(m, [sv], ones16, mask=lane0)
        plsc.store_scatter(m, [tv], ones16, mask=lane0)

        def edge_sweep(body_fn, carry0):
            carry = carry0
            for cidx in range(nchunk):
                pltpu.sync_copy(row_hbm.at[pl.ds(cidx * ch, ch)], er)
                pltpu.sync_copy(col_hbm.at[pl.ds(cidx * ch, ch)], ec)

                def eb(g, cy):
                    r = er[pl.ds(g * 16, 16)]
                    c = ec[pl.ds(g * 16, 16)]
                    return body_fn(r, c, cy)
                carry = lax.fori_loop(0, ch16, eb, carry)
            return carry

        def khop_body(r, c, cy):
            mc = plsc.load_gather(m, [c])
            plsc.addupdate_scatter(acc1, [r], mc)
            return cy

        for _ in range(_NUM_HOPS):
            edge_sweep(khop_body, jnp.int32(0))

            def upd(i, _):
                sl = pl.ds(i * 16, 16)
                m[sl] = jnp.where(acc1[sl] > 0, ones16, m[sl])
                acc1[sl] = zeros16
                return 0
            lax.fori_loop(0, n16, upd, 0)

        def ncomp(i, noff):
            sl = pl.ds(i * 16, 16)
            msk = m[sl] > 0
            ids = jnp.full((16,), i * 16, jnp.int32) + lane
            noffc = jnp.minimum(noff, _CAP - 16)
            plsc.store_compressed(nodeso.at[pl.ds(noffc, 16)], ids, mask=msk)
            return noff + jnp.max(plsc.all_reduce_population_count(msk))

        nvalid_s = lax.fori_loop(0, n16, ncomp, jnp.int32(0))

        fill(dist1, neg16, n16)
        fill(dist2, neg16, n16)
        plsc.store_scatter(dist1, [sv], zeros16, mask=lane0)
        plsc.store_scatter(dist2, [tv], zeros16, mask=lane0)

        ecount_s = jnp.int32(0)
        for d in range(1, _NUM_HOPS + 1):
            dm1 = jnp.full((16,), d - 1, jnp.int32)
            first_hop = d == 1

            def bfs_body(r, c, off, dm1=dm1, first_hop=first_hop):
                mr = plsc.load_gather(m, [r])
                mc = plsc.load_gather(m, [c])
                ev = mr * mc
                dr1 = plsc.load_gather(dist1, [r])
                dc1 = plsc.load_gather(dist1, [c])
                dr2 = plsc.load_gather(dist2, [r])
                dc2 = plsc.load_gather(dist2, [c])
                plsc.addupdate_scatter(acc1, [c], jnp.where(dr1 == dm1, ev, zeros16))
                plsc.addupdate_scatter(acc1, [r], jnp.where(dc1 == dm1, ev, zeros16))
                plsc.addupdate_scatter(acc2, [c], jnp.where(dr2 == dm1, ev, zeros16))
                plsc.addupdate_scatter(acc2, [r], jnp.where(dc2 == dm1, ev, zeros16))
                if first_hop:
                    plsc.addupdate_scatter(degf, [c], ev.astype(jnp.float32))
                    msk = ev > 0
                    offc = jnp.minimum(off, _ECAP - 16)
                    plsc.store_compressed(ero.at[pl.ds(offc, 16)], r, mask=msk)
                    plsc.store_compressed(eco.at[pl.ds(offc, 16)], c, mask=msk)
                    off = off + jnp.max(plsc.all_reduce_population_count(msk))
                return off

            if first_hop:
                ecount_s = edge_sweep(bfs_body, ecount_s)
            else:
                edge_sweep(bfs_body, jnp.int32(0))
            dv = jnp.full((16,), d, jnp.int32)

            def updd(i, _, dv=dv):
                sl = pl.ds(i * 16, 16)
                d1v = dist1[sl]
                dist1[sl] = jnp.where((acc1[sl] > 0) & (d1v < 0), dv, d1v)
                acc1[sl] = zeros16
                d2v = dist2[sl]
                dist2[sl] = jnp.where((acc2[sl] > 0) & (d2v < 0), dv, d2v)
                acc2[sl] = zeros16
                return 0
            lax.fori_loop(0, n16, updd, 0)

        hp = jnp.full((16,), _NUM_HOPS + 1, jnp.int32)

        def labs(i, _):
            sl = pl.ds(i * 16, 16)
            a = dist1[sl]
            b = dist2[sl]
            a = jnp.where(a < 0, hp, a)
            b = jnp.where(b < 0, hp, b)
            mn = jnp.minimum(a, b)
            mx = jnp.minimum(jnp.maximum(a, b), hp)
            acc1[sl] = ones16 + mn + (_NUM_HOPS + 1) * mx
            return 0
        lax.fori_loop(0, n16, labs, 0)

        cntv[...] = jnp.where(lane == 0, nvalid_s,
                              jnp.where(lane == 1, ecount_s, 0))

        pltpu.sync_copy(m, masks_hbm.at[wid])
        pltpu.sync_copy(acc1, labels_hbm.at[wid])
        pltpu.sync_copy(degf, deg_hbm.at[wid])
        pltpu.sync_copy(ero, ero_hbm.at[wid])
        pltpu.sync_copy(eco, eco_hbm.at[wid])
        pltpu.sync_copy(nodeso, nodes_hbm.at[wid])
        pltpu.sync_copy(cntv, cnts_hbm.at[wid])

    return k(row, col, pred.reshape(-1))


def _l1_body(x_ref, at_ref, w_ref, b_ref, vrow_ref, y_ref, s_ref, ss_ref):
    xw = jnp.dot(x_ref[0], w_ref[...], precision=_PREC,
                 preferred_element_type=jnp.float32)
    y = jnp.dot(at_ref[0], xw, precision=_PREC,
                preferred_element_type=jnp.float32)
    y = (y + b_ref[...]) * vrow_ref[0]
    y_ref[0] = y
    s_ref[0, 0] = jnp.sum(y, axis=0)
    ss_ref[0, 0] = jnp.sum(y * y, axis=0)


def _mid_body(y_ref, s_in_ref, ss_in_ref, g_ref, be_ref, ntot_ref,
              at_ref, w_ref, b_ref, vrow_ref, y_out_ref, s_ref, ss_ref):
    ntot = ntot_ref[0, 0]
    S = jnp.sum(s_in_ref[...], axis=0)
    SS = jnp.sum(ss_in_ref[...], axis=0)
    m = S / ntot
    v = (SS - S * S / ntot) / ntot
    scale = g_ref[0] / jnp.sqrt(v + 1e-5)
    shift = be_ref[0] - m * scale
    vr = vrow_ref[0]
    z = jnp.maximum((y_ref[0] * scale + shift) * vr, 0.0)
    xw = jnp.dot(z, w_ref[...], precision=_PREC,
                 preferred_element_type=jnp.float32)
    y = (jnp.dot(at_ref[0], xw, precision=_PREC,
                 preferred_element_type=jnp.float32) + b_ref[...]) * vr
    y_out_ref[0] = y
    s_ref[0, 0] = jnp.sum(y, axis=0)
    ss_ref[0, 0] = jnp.sum(y * y, axis=0)


def _readout_body(s3_ref, ss3_ref, cnt_ref, ntot_ref, g_ref, be_ref,
                  wc1_ref, bc1_ref, wc2_ref, bc2_ref, out_ref):
    ntot = ntot_ref[0, 0]
    S = jnp.sum(s3_ref[...], axis=0)
    SS = jnp.sum(ss3_ref[...], axis=0)
    m = S / ntot
    v = (SS - S * S / ntot) / ntot
    scale = g_ref[0] / jnp.sqrt(v + 1e-5)
    shift = be_ref[0] - m * scale
    cnt = cnt_ref[...]
    pooled = (s3_ref[...] * scale + shift * cnt) / cnt
    h = jnp.maximum(jnp.dot(pooled, wc1_ref[...], precision=_PREC,
                            preferred_element_type=jnp.float32) + bc1_ref[...],
                    0.0)
    out_ref[...] = jnp.dot(h, wc2_ref[...], precision=_PREC,
                           preferred_element_type=jnp.float32) + bc2_ref[...]


def _fast_path(x_cp, AT, vrow, cnt, n_tot, W1p, b1, W2, b2, W3, b3,
               g1, be1, g2, be2, g3, be3, Wc1, bc1, Wc2p, bc2p):
    G = x_cp.shape[0]
    H = W1p.shape[1]
    full = lambda shape: pl.BlockSpec(shape, lambda j: (0,) * len(shape))
    per_g3 = lambda shape: pl.BlockSpec(shape, lambda j: (j, 0, 0))
    vrow3 = vrow[:, :, None]

    y1, s1, ss1 = _pallas_call(
        _l1_body,
        grid=(G,),
        in_specs=[per_g3((1, _CAP, 256)), per_g3((1, _CAP, _CAP)),
                  full((256, H)), full((1, H)), per_g3((1, _CAP, 1))],
        out_specs=[per_g3((1, _CAP, H)), per_g3((1, 1, H)), per_g3((1, 1, H))],
        out_shape=[jax.ShapeDtypeStruct((G, _CAP, H), jnp.float32),
                   jax.ShapeDtypeStruct((G, 1, H), jnp.float32),
                   jax.ShapeDtypeStruct((G, 1, H), jnp.float32)],
    )(x_cp, AT, W1p, b1, vrow3)

    def mid(y, s_in, ss_in, g, be, W, b):
        return _pallas_call(
            _mid_body,
            grid=(G,),
            in_specs=[per_g3((1, _CAP, H)), full((G, H)), full((G, H)),
                      full((1, H)), full((1, H)), full((1, 1)),
                      per_g3((1, _CAP, _CAP)), full((H, H)), full((1, H)),
                      per_g3((1, _CAP, 1))],
            out_specs=[per_g3((1, _CAP, H)), per_g3((1, 1, H)), per_g3((1, 1, H))],
            out_shape=[jax.ShapeDtypeStruct((G, _CAP, H), jnp.float32),
                       jax.ShapeDtypeStruct((G, 1, H), jnp.float32),
                       jax.ShapeDtypeStruct((G, 1, H), jnp.float32)],
        )(y, s_in.reshape(G, H), ss_in.reshape(G, H), g, be, n_tot, AT, W, b,
          vrow3)

    y2, s2, ss2 = mid(y1, s1, ss1, g1, be1, W2, b2)
    y3, s3, ss3 = mid(y2, s2, ss2, g2, be2, W3, b3)
    s3 = s3.reshape(G, H)
    ss3 = ss3.reshape(G, H)

    full0 = lambda shape: pl.BlockSpec(shape, lambda: (0,) * len(shape))
    out = _pallas_call(
        _readout_body,
        in_specs=[full0((G, H)), full0((G, H)), full0((G, 1)), full0((1, 1)),
                  full0((1, H)), full0((1, H)), full0((H, 128)), full0((1, 128)),
                  full0((128, 128)), full0((1, 128))],
        out_specs=full0((G, 128)),
        out_shape=jax.ShapeDtypeStruct((G, 128), jnp.float32),
    )(s3, ss3, cnt, n_tot, g3, be3, Wc1, bc1, Wc2p, bc2p)
    return out[:, 0]


def kernel(edge_index, node_features, pred_edges, emb_table, W1, b1, W2, b2,
           W3, b3, g1, be1, g2, be2, g3, be3, Wc1, bc1, Wc2, bc2):
    num_nodes = node_features.shape[0]
    num_graphs = pred_edges.shape[1]
    feat = node_features.shape[1]
    row, col = edge_index[0], edge_index[1]

    num_edges = row.shape[0]
    use_sc = (num_graphs == 32 and num_nodes % 16 == 0
              and num_edges % 16000 == 0)
    if use_sc:
        masks_i, labels, degs, row_e, col_e, nodes, cnts = _phase1_sc(
            row, col, pred_edges, num_nodes, num_edges, num_graphs)
        masks = masks_i.astype(bool)
        evalid = None
        deg = degs + 1.0
        nvalid = cnts[:, 0]
        ecount = cnts[:, 1]
    else:
        def khop_mask(s, t):
            m = jnp.zeros((num_nodes,), dtype=bool).at[s].set(True).at[t].set(True)
            for _ in range(_NUM_HOPS):
                hit = jnp.zeros((num_nodes,), dtype=jnp.int32).at[row].add(
                    m[col].astype(jnp.int32))
                m = m | (hit > 0)
            return m

        masks = jax.vmap(khop_mask)(pred_edges[0], pred_edges[1])
        evalid = masks[:, row] & masks[:, col]

        def bfs(start, ev):
            dist = jnp.full((num_nodes,), -1, dtype=jnp.int32).at[start].set(0)
            frontier = jnp.zeros((num_nodes,), dtype=bool).at[start].set(True)
            visited = frontier
            for d in range(1, _NUM_HOPS + 1):
                fwd = jnp.zeros((num_nodes,), dtype=jnp.int32).at[col].add(
                    (frontier[row] & ev).astype(jnp.int32))
                bwd = jnp.zeros((num_nodes,), dtype=jnp.int32).at[row].add(
                    (frontier[col] & ev).astype(jnp.int32))
                nxt = ((fwd + bwd) > 0) & (~visited)
                dist = jnp.where(nxt, d, dist)
                visited = visited | nxt
                frontier = nxt
            return dist

        d1 = jax.vmap(bfs)(pred_edges[0], evalid)
        d2 = jax.vmap(bfs)(pred_edges[1], evalid)
        hp1 = _NUM_HOPS + 1
        d1 = jnp.where(d1 == -1, hp1, d1)
        d2 = jnp.where(d2 == -1, hp1, d2)
        labels = 1 + jnp.minimum(d1, d2) + hp1 * jnp.minimum(
            jnp.maximum(d1, d2), hp1)

        ef = evalid.astype(jnp.float32)
        deg = jax.vmap(
            lambda e: jnp.zeros((num_nodes,), jnp.float32).at[col].add(e))(ef) + 1.0

        nvalid = jnp.sum(masks.astype(jnp.int32), axis=1)
        nodes = jax.vmap(
            lambda mm: jnp.nonzero(mm, size=_CAP, fill_value=num_nodes)[0])(masks)
        nodes = nodes.astype(jnp.int32)
        ecount = jnp.sum(evalid.astype(jnp.int32), axis=1)
        eidx = jax.vmap(
            lambda e: jnp.nonzero(e, size=_ECAP, fill_value=0)[0])(evalid)
        eidx = eidx.astype(jnp.int32)
        row_e = row[eidx]
        col_e = col[eidx]

    dinv = jax.lax.rsqrt(deg)

    validrow = (jnp.arange(_CAP)[None, :] < nvalid[:, None]).astype(jnp.float32)

    inv = jax.vmap(lambda nd: jnp.full((num_nodes + 1,), _CAP, jnp.int32)
                   .at[nd].set(jnp.arange(_CAP, dtype=jnp.int32)))(nodes)

    eflag = (jnp.arange(_ECAP)[None, :] < ecount[:, None]).astype(jnp.float32)

    norm = (jnp.take_along_axis(dinv, row_e, axis=1)
            * jnp.take_along_axis(dinv, col_e, axis=1) * eflag)
    r_c = jnp.take_along_axis(inv, row_e, axis=1)
    c_c = jnp.take_along_axis(inv, col_e, axis=1)

    dinv_pad = jnp.concatenate([dinv, jnp.ones((num_graphs, 1), jnp.float32)], 1)
    dvn = jnp.take_along_axis(dinv_pad, nodes, axis=1)
    dsq = dvn * dvn * validrow

    ar = jnp.arange(_CAP)

    def build_at(cc, rc, nm, ds):
        A = jnp.zeros((_CAP, _CAP), jnp.float32).at[cc, rc].add(
            nm, mode='drop')
        return A.at[ar, ar].add(ds)

    AT = jax.vmap(build_at)(c_c, r_c, norm, dsq)

    nf_pad = jnp.concatenate(
        [node_features, jnp.zeros((1, feat), jnp.float32)], 0)
    nf_c = nf_pad[nodes]
    lab_pad = jnp.concatenate(
        [labels, jnp.zeros((num_graphs, 1), labels.dtype)], 1)
    lab_c = jnp.take_along_axis(lab_pad, nodes, axis=1)
    emb_c = emb_table[lab_c]
    x_c = jnp.concatenate([nf_c, emb_c], -1) * validrow[..., None]
    in_dim = x_c.shape[-1]
    x_cp = jnp.pad(x_c, ((0, 0), (0, 0), (0, 256 - in_dim)))
    W1p = jnp.pad(W1, ((0, 256 - in_dim), (0, 0)))

    cnt = nvalid.astype(jnp.float32)[:, None]
    n_tot = jnp.sum(cnt).reshape(1, 1)
    Wc2p = jnp.pad(Wc2, ((0, 0), (0, 127)))
    bc2p = jnp.pad(bc2, (0, 127))[None, :]

    ok = (jnp.max(nvalid) <= _CAP) & (jnp.max(ecount) <= _ECAP)

    def fast():
        return _fast_path(x_cp, AT, validrow, cnt, n_tot, W1p, b1[None, :],
                          W2, b2[None, :], W3, b3[None, :], g1[None, :],
                          be1[None, :], g2[None, :], be2[None, :], g3[None, :],
                          be3[None, :], Wc1, bc1[None, :], Wc2p, bc2p)

    def slow():
        ev_l = (masks[:, row] & masks[:, col]) if evalid is None else evalid
        mf = masks.astype(node_features.dtype)
        x = jnp.concatenate([
            jnp.broadcast_to(node_features,
                             (num_graphs,) + node_features.shape),
            emb_table[labels]], axis=-1) * mf[..., None]
        ntot_s = jnp.sum(mf)

        def gcn(x, W, b):
            def per_graph(args):
                xj, evj, mj = args
                xw = xj @ W
                efj = evj.astype(xj.dtype)
                degj = jnp.zeros((num_nodes,), dtype=xj.dtype).at[col].add(efj) + 1.0
                dinvj = 1.0 / jnp.sqrt(jnp.clip(degj, 1.0))
                normj = dinvj[row] * dinvj[col] * efj
                outj = jnp.zeros((num_nodes, W.shape[1]), dtype=xj.dtype).at[col].add(
                    xw[row] * normj[:, None])
                outj = outj + xw * (dinvj * dinvj)[:, None]
                return (outj + b) * mj[:, None]
            return jax.lax.map(per_graph, (x, ev_l, mf))

        def bn(x, g, be):
            m = jnp.sum(x, axis=(0, 1)) / ntot_s
            v = jnp.sum(jnp.square(x - m) * mf[..., None], axis=(0, 1)) / ntot_s
            return ((x - m) / jnp.sqrt(v + 1e-5) * g + be) * mf[..., None]

        xs = x
        for i, (W, b, g, be) in enumerate(
                [(W1, b1, g1, be1), (W2, b2, g2, be2), (W3, b3, g3, be3)]):
            xs = gcn(xs, W, b)
            xs = bn(xs, g, be)
            if i < 2:
                xs = jax.nn.relu(xs)
        counts = jnp.sum(mf, axis=1)
        pooled = jnp.sum(xs, axis=1) / counts[:, None]
        h = jax.nn.relu(pooled @ Wc1 + bc1)
        return (h @ Wc2 + bc2).squeeze(-1)

    return jax.lax.cond(ok, fast, slow)
